# baseline (speedup 1.0000x reference)
import functools

import jax
import jax.numpy as jnp
from jax.experimental import pallas as pl
from jax.experimental.pallas import tpu as pltpu

KNN = 20
EPS = 1e-5


def _lrelu(x):
    return jnp.where(x > 0, x, 0.2 * x)


def _bn(x, gamma, beta, axes):
    mean = jnp.mean(x, axis=axes, keepdims=True)
    var = jnp.mean((x - mean) ** 2, axis=axes, keepdims=True)
    xn = (x - mean) / jnp.sqrt(var + EPS)
    shp = [1] * x.ndim
    shp[1] = -1
    return xn * gamma.reshape(shp) + beta.reshape(shp)


def _matmul_body(w_ref, x_ref, o_ref):
    o_ref[...] = jax.lax.dot_general(
        w_ref[...], x_ref[...], (((1,), (0,)), ((), ())),
        preferred_element_type=jnp.float32)


def _pallas_conv1d(W, x):
    B, C, N = x.shape
    O = W.shape[0]
    X = jnp.transpose(x, (1, 0, 2)).reshape(C, B * N)
    TN = 2048
    out = pl.pallas_call(
        _matmul_body,
        grid=(B * N // TN,),
        in_specs=[pl.BlockSpec((O, C), lambda i: (0, 0)),
                  pl.BlockSpec((C, TN), lambda i: (0, i))],
        out_specs=pl.BlockSpec((O, TN), lambda i: (0, i)),
        out_shape=jax.ShapeDtypeStruct((O, B * N), jnp.float32),
    )(W, X)
    return jnp.transpose(out.reshape(O, B, N), (1, 0, 2))


def _knn_idx(x, k):
    inner = -2.0 * jnp.einsum('bcn,bcm->bnm', x, x)
    xx = jnp.sum(x ** 2, axis=1, keepdims=True)
    pd = -xx - inner - jnp.transpose(xx, (0, 2, 1))
    return jax.lax.top_k(pd, k)[1]


def _graph_feature(x, k):
    B, C, N = x.shape
    idx = _knn_idx(x, k)
    xt = jnp.transpose(x, (0, 2, 1))
    nb = xt[jnp.arange(B)[:, None, None], idx]
    xc = jnp.broadcast_to(xt[:, :, None, :], (B, N, k, C))
    feat = jnp.concatenate([nb - xc, xc], axis=3)
    return jnp.transpose(feat, (0, 3, 1, 2))


def _conv2d(W, x):
    return jnp.einsum('oi,bink->bonk', W, x)


def kernel(x, W1, g1, b1, W2, g2, b2, W3, g3, b3, W4, g4, b4, W5, g5, b5,
           W6, g6, b6, W7, g7, b7, W8, b8):
    N = x.shape[2]
    x1 = _lrelu(_bn(_pallas_conv1d(W1, x), g1, b1, (0, 2)))
    f = _graph_feature(x1, KNN)
    x2 = jnp.max(_lrelu(_bn(_conv2d(W2, f), g2, b2, (0, 2, 3))), axis=-1)
    f = _graph_feature(x2, KNN)
    x3 = jnp.max(_lrelu(_bn(_conv2d(W3, f), g3, b3, (0, 2, 3))), axis=-1)
    x6 = _lrelu(_bn(_pallas_conv1d(W4, x3), g4, b4, (0, 2)))
    x6 = jnp.max(x6, axis=-1, keepdims=True)
    x6 = jnp.broadcast_to(x6, (x6.shape[0], x6.shape[1], N))
    x7 = jnp.concatenate([x1, x2, x3, x6], axis=1)
    x7 = _lrelu(_bn(_pallas_conv1d(W5, x7), g5, b5, (0, 2)))
    x7 = _lrelu(_bn(_pallas_conv1d(W6, x7), g6, b6, (0, 2)))
    x7 = _lrelu(_bn(_pallas_conv1d(W7, x7), g7, b7, (0, 2)))
    return _pallas_conv1d(W8, x7) + b8.reshape(1, -1, 1)
